# baseline (speedup 1.0000x reference)
import functools

import jax
import jax.numpy as jnp
from jax import lax
from jax.experimental import pallas as pl
from jax.experimental.pallas import tpu as pltpu
from jax.experimental.pallas import tpu_sc as plsc

NC = 2
NS = 16
NW = NC * NS
LANES = 128
ZR = 128
NPAD = NS * 5 * ZR


def _worker_id():
    return lax.axis_index("s") * NC + lax.axis_index("c")


def _sc_degree(src_rows, dst_rows, n):
    R = dst_rows.shape[0]
    pw = R // NW
    mesh = plsc.VectorSubcoreMesh(core_axis_name="c", subcore_axis_name="s")

    @functools.partial(
        pl.kernel,
        out_type=jax.ShapeDtypeStruct((NC * NPAD,), jnp.float32),
        mesh=mesh,
        scratch_types=[
            pltpu.VMEM((pw, LANES), jnp.int32),
            pltpu.VMEM((LANES,), jnp.float32),
            pltpu.VMEM((2048,), jnp.float32),
            pltpu.VMEM_SHARED((NPAD,), jnp.float32),
            pltpu.SemaphoreType.DMA,
        ],
    )
    def k(dst_hbm, out_hbm, idx_v, ones_v, zb_v, acc_sh, sem):
        cid = lax.axis_index("c")
        sid = lax.axis_index("s")
        wid = _worker_id()

        def fill_z(i, _):
            zb_v[pl.ds(i * 16, 16)] = jnp.zeros((16,), jnp.float32)
            return 0
        lax.fori_loop(0, 128, fill_z, 0)
        for j in range(LANES // 16):
            ones_v[pl.ds(j * 16, 16)] = jnp.ones((16,), jnp.float32)

        @pl.when(sid < 5)
        def _():
            pltpu.sync_copy(zb_v, acc_sh.at[pl.ds(sid * 2048, 2048)])
        pltpu.sync_copy(dst_hbm.at[pl.ds(wid * pw, pw)], idx_v)
        plsc.subcore_barrier()

        K = 16
        @pl.loop(0, pw, step=K)
        def _(t):
            for s in range(K):
                pltpu.async_copy(
                    ones_v, acc_sh.at[idx_v.at[t + s]], sem, add=True)
            for s in range(K):
                pltpu.make_async_copy(
                    ones_v, acc_sh.at[idx_v.at[t]], sem).wait()
        plsc.subcore_barrier()

        @pl.when(sid < 10)
        def _():
            pltpu.sync_copy(acc_sh.at[pl.ds(sid * 1024, 1024)],
                            zb_v.at[pl.ds(0, 1024)])
            pltpu.sync_copy(zb_v.at[pl.ds(0, 1024)],
                            out_hbm.at[pl.ds(cid * NPAD + sid * 1024, 1024)])

    return k(dst_rows)


def _sc_aggregate(src_rows, dst_rows, g):
    R = src_rows.shape[0]
    C = g.shape[1]
    pw = R // NW
    mesh = plsc.VectorSubcoreMesh(core_axis_name="c", subcore_axis_name="s")

    @functools.partial(
        pl.kernel,
        out_type=jax.ShapeDtypeStruct((NC, NPAD, C), jnp.float32),
        mesh=mesh,
        scratch_types=[
            pltpu.VMEM((2, 2, LANES), jnp.int32),
            pltpu.VMEM((2, 2, LANES), jnp.int32),
            pltpu.VMEM((2, LANES, C), jnp.float32),
            pltpu.VMEM_SHARED((NPAD, C), jnp.float32),
        ] + [pltpu.SemaphoreType.DMA] * 6,
    )
    def k(src_hbm, dst_hbm, g_hbm, out_hbm, idx_v, idx2_v, rows_v, acc_sh,
          *sems):
        gsem = sems[:2]
        ssem = sems[2:4]
        isem = sems[4:]
        cid = lax.axis_index("c")
        sid = lax.axis_index("s")
        wid = _worker_id()
        base = wid * pw

        def load_idx(row, buf, slot, sem):
            pltpu.async_copy(src_hbm.at[base + row], buf.at[slot, 0], sem)
            pltpu.async_copy(dst_hbm.at[base + row], buf.at[slot, 1], sem)

        def wait_idx(slot, sem):
            pltpu.make_async_copy(src_hbm.at[base], idx2_v.at[slot, 0],
                                  sem).wait()
            pltpu.make_async_copy(dst_hbm.at[base], idx2_v.at[slot, 1],
                                  sem).wait()

        def fill_z(i, _):
            for j in range(C // 16):
                rows_v[0, i, pl.ds(j * 16, 16)] = jnp.zeros(
                    (16,), jnp.float32)
            return 0
        lax.fori_loop(0, ZR, fill_z, 0)
        for t in range(5):
            pltpu.sync_copy(rows_v.at[0],
                            acc_sh.at[pl.ds((sid * 5 + t) * ZR, ZR)])
        pltpu.sync_copy(src_hbm.at[base], idx_v.at[0, 0])
        pltpu.sync_copy(dst_hbm.at[base], idx_v.at[0, 1])
        plsc.subcore_barrier()

        pltpu.async_copy(g_hbm.at[idx_v.at[0, 0]], rows_v.at[0], gsem[0])
        load_idx(1, idx2_v, 1, isem[1])
        load_idx(2, idx2_v, 0, isem[0])

        @pl.loop(0, pw, step=2)
        def _(t):
            for b in range(2):
                r = t + b
                pltpu.make_async_copy(
                    g_hbm.at[idx_v.at[b, 0]], rows_v.at[b], gsem[b]).wait()
                pltpu.async_copy(
                    rows_v.at[b], acc_sh.at[idx_v.at[b, 1]], ssem[b],
                    add=True)
                o = 1 - b

                @pl.when(r + 1 < pw)
                def _():
                    @pl.when(r > 0)
                    def _():
                        pltpu.make_async_copy(
                            rows_v.at[o], acc_sh.at[idx_v.at[o, 1]],
                            ssem[o]).wait()
                    wait_idx(o, isem[o])
                    for d in range(2):
                        for j in range(LANES // 16):
                            idx_v[o, d, pl.ds(j * 16, 16)] = \
                                idx2_v[o, d, pl.ds(j * 16, 16)]
                    pltpu.async_copy(
                        g_hbm.at[idx_v.at[o, 0]], rows_v.at[o], gsem[o])

                    @pl.when(r + 3 < pw)
                    def _():
                        load_idx(r + 3, idx2_v, o, isem[o])

        for b in range(2):
            pltpu.make_async_copy(
                rows_v.at[b], acc_sh.at[idx_v.at[b, 1]], ssem[b]).wait()
        plsc.subcore_barrier()

        for t in range(5):
            r0 = (sid * 5 + t) * ZR
            pltpu.sync_copy(acc_sh.at[pl.ds(r0, ZR)], rows_v.at[0])
            pltpu.sync_copy(rows_v.at[0], out_hbm.at[cid, pl.ds(r0, ZR)])

    return k(src_rows, dst_rows, g)


def _tc_stage_a(x, W1, bn1_gamma, bn1_beta, deg_part):
    n, C = x.shape

    def body(x_ref, w_ref, gam_ref, bet_ref, degp_ref, g_ref, dinv_ref):
        xv = x_ref[...]
        mean = jnp.mean(xv, axis=0, keepdims=True)
        xc = xv - mean
        var = jnp.mean(xc * xc, axis=0, keepdims=True)
        xn = xc * lax.rsqrt(var + 1e-5) * gam_ref[...][None, :] \
            + bet_ref[...][None, :]
        deg = degp_ref[0, :n] + degp_ref[1, :n] + 1.0
        dinv = lax.rsqrt(deg)
        m = jnp.dot(xn, w_ref[...], preferred_element_type=jnp.float32)
        g_ref[:n] = m * dinv
        g_ref[n:] = jnp.zeros((NPAD - n, C), jnp.float32)
        dinv_ref[...] = dinv

    return pl.pallas_call(
        body,
        out_shape=(jax.ShapeDtypeStruct((NPAD, C), jnp.float32),
                   jax.ShapeDtypeStruct((n, 1), jnp.float32)),
    )(x, W1, bn1_gamma, bn1_beta, deg_part)


def _tc_stage_b(s_part, g1, dinv, b1, bn2_gamma, bn2_beta, W2):
    n = dinv.shape[0]
    C = g1.shape[1]

    def body(s_ref, g1_ref, dinv_ref, b1_ref, gam_ref, bet_ref, w_ref,
             g2_ref):
        dinv = dinv_ref[...]
        s = s_ref[0, :n] + s_ref[1, :n]
        h = (s + g1_ref[:n]) * dinv + b1_ref[...][None, :]
        h = jnp.maximum(h, 0.0)
        mean = jnp.mean(h, axis=0, keepdims=True)
        hc = h - mean
        var = jnp.mean(hc * hc, axis=0, keepdims=True)
        hn = hc * lax.rsqrt(var + 1e-5) * gam_ref[...][None, :] \
            + bet_ref[...][None, :]
        m = jnp.dot(hn, w_ref[...], preferred_element_type=jnp.float32)
        g2_ref[:n] = m * dinv
        g2_ref[n:] = jnp.zeros((NPAD - n, C), jnp.float32)

    return pl.pallas_call(
        body,
        out_shape=jax.ShapeDtypeStruct((NPAD, C), jnp.float32),
    )(s_part, g1, dinv, b1, bn2_gamma, bn2_beta, W2)


def _tc_stage_c(s_part, g2, dinv, b2, x):
    n, C = x.shape

    def body(s_ref, g2_ref, dinv_ref, b2_ref, x_ref, out_ref):
        s = s_ref[0, :n] + s_ref[1, :n]
        h = (s + g2_ref[:n]) * dinv_ref[...] + b2_ref[...][None, :]
        out_ref[...] = jnp.maximum(h, 0.0) + x_ref[...]

    return pl.pallas_call(
        body,
        out_shape=jax.ShapeDtypeStruct((n, C), jnp.float32),
    )(s_part, g2, dinv, b2, x)


def kernel(x, edge_index, bn1_gamma, bn1_beta, W1, b1,
           bn2_gamma, bn2_beta, W2, b2):
    n, C = x.shape
    E = edge_index.shape[1]
    R = E // LANES
    blk = NW * 2
    Rpad = ((R + blk - 1) // blk) * blk
    ei = edge_index.reshape(2, R, LANES)
    if Rpad > R:
        padv = (n + (jnp.arange((Rpad - R) * LANES, dtype=jnp.int32) % 240)
                ).reshape(1, Rpad - R, LANES)
        ei = jnp.concatenate(
            [ei, jnp.broadcast_to(padv, (2, Rpad - R, LANES))], axis=1)
    src_rows = ei[0]
    dst_rows = ei[1]

    deg_part = _sc_degree(src_rows, dst_rows, n).reshape(NC, NPAD, 1)
    g1, dinv = _tc_stage_a(x, W1, bn1_gamma, bn1_beta, deg_part)
    s1 = _sc_aggregate(src_rows, dst_rows, g1)
    g2 = _tc_stage_b(s1, g1, dinv, b1, bn2_gamma, bn2_beta, W2)
    s2 = _sc_aggregate(src_rows, dst_rows, g2)
    return _tc_stage_c(s2, g2, dinv, b2, x)
